# baseline (speedup 1.0000x reference)
import functools

import jax
import jax.numpy as jnp
import numpy as np
from jax import lax
from jax.experimental import pallas as pl
from jax.experimental.pallas import tpu as pltpu
from jax.experimental.pallas import tpu_sc as plsc

NUM_LABELS = 64
B, H, W = 4, 512, 512
NC, NS, L = 2, 16, 16
NW = NC * NS

SC_IMGS = 1
ROWS = SC_IMGS * H // NW
RC = ROWS // 2

_LO = np.int32(np.uint32(0x728BBBAF))
_HI = np.int32(np.uint32(0x4C65DA36))


@functools.partial(
    pl.kernel,
    mesh=plsc.VectorSubcoreMesh(core_axis_name="c", subcore_axis_name="s"),
    out_type=jax.ShapeDtypeStruct((SC_IMGS, H, W), jnp.float32),
    scratch_types=[
        pltpu.VMEM((2, RC, W), jnp.float32),
        pltpu.VMEM((2, RC, W), jnp.int32),
        pltpu.SemaphoreType.DMA,
        pltpu.SemaphoreType.DMA,
        pltpu.SemaphoreType.DMA,
    ],
)
def _sc_clear(inp_hbm, lab_hbm, out_hbm, inp_v, lab_v, s0, s1, so):
    wid = lax.axis_index("s") * NC + lax.axis_index("c")
    img = wid // (H // ROWS)
    r0 = (wid % (H // ROWS)) * ROWS
    lo = jnp.full((L,), _LO, dtype=jnp.int32)
    hi = jnp.full((L,), _HI, dtype=jnp.int32)

    sems = (s0, s1)
    in_copies = []
    for k in range(2):
        in_copies.append(
            (
                pltpu.async_copy(
                    inp_hbm.at[img, pl.ds(r0 + k * RC, RC)], inp_v.at[k], sems[k]
                ),
                pltpu.async_copy(
                    lab_hbm.at[img, pl.ds(r0 + k * RC, RC)], lab_v.at[k], sems[k]
                ),
            )
        )
    out_copies = []
    for k in range(2):
        for c in in_copies[k]:
            c.wait()

        @plsc.parallel_loop(0, RC * W, step=L, unroll=4)
        def _loop(off):
            r = off // W
            c = off % W
            labs = lab_v[k, r, pl.ds(c, L)]
            word = jnp.where(labs < 32, lo, hi)
            keep = (word >> (labs & 31)) & 1
            x = inp_v[k, r, pl.ds(c, L)]
            inp_v[k, r, pl.ds(c, L)] = jnp.where(keep == 1, x, 0.0)

        out_copies.append(
            pltpu.async_copy(
                inp_v.at[k], out_hbm.at[img, pl.ds(r0 + k * RC, RC)], so
            )
        )
    for c in out_copies:
        c.wait()


def _tc_body(inp_ref, lab_ref, out_ref):
    labs = lab_ref[...]
    word = jnp.where(labs < 32, jnp.int32(_LO), jnp.int32(_HI))
    keep = (word >> (labs & 31)) & 1
    out_ref[...] = jnp.where(keep == 1, inp_ref[...], 0.0)


_TC_BR = 128

_tc_clear = pl.pallas_call(
    _tc_body,
    grid=(B - SC_IMGS, H // _TC_BR),
    in_specs=[
        pl.BlockSpec((1, _TC_BR, W), lambda b, i: (b + SC_IMGS, i, 0)),
        pl.BlockSpec((1, _TC_BR, W), lambda b, i: (b + SC_IMGS, i, 0)),
    ],
    out_specs=pl.BlockSpec((1, _TC_BR, W), lambda b, i: (b, i, 0)),
    out_shape=jax.ShapeDtypeStruct((B - SC_IMGS, H, W), jnp.float32),
)


def kernel(input_tensor, label_tensor):
    sc_out = _sc_clear(input_tensor, label_tensor)
    tc_out = _tc_clear(input_tensor, label_tensor)
    out = jnp.concatenate([sc_out, tc_out], axis=0)
    return out


# speedup vs baseline: 1.1433x; 1.1433x over previous
import functools

import jax
import jax.numpy as jnp
import numpy as np
from jax import lax
from jax.experimental import pallas as pl
from jax.experimental.pallas import tpu as pltpu
from jax.experimental.pallas import tpu_sc as plsc

NUM_LABELS = 64
B, H, W = 4, 512, 512
NC, NS, L = 2, 16, 16
NW = NC * NS

BANDS = B * H // NW
NCK = 4
RC = BANDS // NCK

_LO = np.int32(np.uint32(0x728BBBAF))
_HI = np.int32(np.uint32(0x4C65DA36))


@functools.partial(
    pl.kernel,
    mesh=plsc.VectorSubcoreMesh(core_axis_name="c", subcore_axis_name="s"),
    out_type=jax.ShapeDtypeStruct((B, H, W), jnp.float32),
    scratch_types=[
        pltpu.VMEM((NCK, RC, W), jnp.float32),
        pltpu.VMEM((NCK, RC, W), jnp.int32),
    ]
    + [pltpu.SemaphoreType.DMA] * (2 * NCK),
)
def _sc_clear(inp_hbm, lab_hbm, out_hbm, inp_v, lab_v, *sems):
    wid = lax.axis_index("s") * NC + lax.axis_index("c")
    img = wid // (H // BANDS)
    r0 = (wid % (H // BANDS)) * BANDS
    lo = jnp.full((L,), _LO, dtype=jnp.int32)
    hi = jnp.full((L,), _HI, dtype=jnp.int32)

    in_sems = sems[:NCK]
    out_sems = sems[NCK:]
    in_copies = []
    for k in range(NCK):
        in_copies.append(
            (
                pltpu.async_copy(
                    inp_hbm.at[img, pl.ds(r0 + k * RC, RC)],
                    inp_v.at[k],
                    in_sems[k],
                ),
                pltpu.async_copy(
                    lab_hbm.at[img, pl.ds(r0 + k * RC, RC)],
                    lab_v.at[k],
                    in_sems[k],
                ),
            )
        )
    out_copies = []
    for k in range(NCK):
        for c in in_copies[k]:
            c.wait()

        @plsc.parallel_loop(0, RC * W, step=L, unroll=4)
        def _loop(off):
            r = off // W
            c = off % W
            labs = lab_v[k, r, pl.ds(c, L)]
            word = jnp.where(labs < 32, lo, hi)
            keep = (word >> (labs & 31)) & 1
            x = inp_v[k, r, pl.ds(c, L)]
            inp_v[k, r, pl.ds(c, L)] = jnp.where(keep == 1, x, 0.0)

        out_copies.append(
            pltpu.async_copy(
                inp_v.at[k],
                out_hbm.at[img, pl.ds(r0 + k * RC, RC)],
                out_sems[k],
            )
        )
    for c in out_copies:
        c.wait()


def kernel(input_tensor, label_tensor):
    return _sc_clear(input_tensor, label_tensor)


# speedup vs baseline: 1.1582x; 1.0130x over previous
import functools

import jax
import jax.numpy as jnp
import numpy as np
from jax import lax
from jax.experimental import pallas as pl
from jax.experimental.pallas import tpu as pltpu
from jax.experimental.pallas import tpu_sc as plsc

NUM_LABELS = 64
B, H, W = 4, 512, 512
NC, NS, L = 2, 16, 16
NW = NC * NS

BANDS = B * H // NW
NCK = 8
RC = BANDS // NCK

_LO = np.int32(np.uint32(0x728BBBAF))
_HI = np.int32(np.uint32(0x4C65DA36))


@functools.partial(
    pl.kernel,
    mesh=plsc.VectorSubcoreMesh(core_axis_name="c", subcore_axis_name="s"),
    out_type=jax.ShapeDtypeStruct((B, H, W), jnp.float32),
    scratch_types=[
        pltpu.VMEM((NCK, RC, W), jnp.float32),
        pltpu.VMEM((NCK, RC, W), jnp.int32),
    ]
    + [pltpu.SemaphoreType.DMA] * (2 * NCK),
)
def _sc_clear(inp_hbm, lab_hbm, out_hbm, inp_v, lab_v, *sems):
    wid = lax.axis_index("s") * NC + lax.axis_index("c")
    img = wid // (H // BANDS)
    r0 = (wid % (H // BANDS)) * BANDS
    lo = jnp.full((L,), _LO, dtype=jnp.int32)
    hi = jnp.full((L,), _HI, dtype=jnp.int32)

    in_sems = sems[:NCK]
    out_sems = sems[NCK:]
    in_copies = []
    for k in range(NCK):
        in_copies.append(
            (
                pltpu.async_copy(
                    inp_hbm.at[img, pl.ds(r0 + k * RC, RC)],
                    inp_v.at[k],
                    in_sems[k],
                ),
                pltpu.async_copy(
                    lab_hbm.at[img, pl.ds(r0 + k * RC, RC)],
                    lab_v.at[k],
                    in_sems[k],
                ),
            )
        )
    out_copies = []
    for k in range(NCK):
        for c in in_copies[k]:
            c.wait()

        @plsc.parallel_loop(0, RC * W, step=L, unroll=8)
        def _loop(off):
            r = off // W
            c = off % W
            labs = lab_v[k, r, pl.ds(c, L)]
            word = jnp.where(labs < 32, lo, hi)
            keep = (word >> (labs & 31)) & 1
            x = inp_v[k, r, pl.ds(c, L)]
            inp_v[k, r, pl.ds(c, L)] = jnp.where(keep == 1, x, 0.0)

        out_copies.append(
            pltpu.async_copy(
                inp_v.at[k],
                out_hbm.at[img, pl.ds(r0 + k * RC, RC)],
                out_sems[k],
            )
        )
    for c in out_copies:
        c.wait()


def kernel(input_tensor, label_tensor):
    return _sc_clear(input_tensor, label_tensor)
